# baseline (speedup 1.0000x reference)
import jax
import jax.numpy as jnp
from jax.experimental import pallas as pl
from jax.experimental.pallas import tpu as pltpu

_GAMMA = 1.5


def _qfocal_body(pred_ref, label_ref, score_ref, out_ref):
    x = pred_ref[0]
    sc = score_ref[0]
    lab = label_ref[0]

    e = jnp.exp(-jnp.abs(x))
    recip = 1.0 / (1.0 + e)
    sig = jnp.where(x >= 0.0, recip, e * recip)
    bce0 = jnp.maximum(x, 0.0) + jnp.log1p(e)
    neg = bce0 * sig * jnp.sqrt(sig)

    s = jnp.max(sc, axis=0, keepdims=True)
    d = jnp.abs(s - sig)
    pos = (bce0 - x * s) * d * jnp.sqrt(d)

    cid = jax.lax.broadcasted_iota(jnp.int32, x.shape, 0)
    m = cid == lab
    out_ref[0] = jnp.where(m, pos, neg)


def kernel(pred, label, score):
    B, N, C = pred.shape
    bn = 1024
    nb = N // bn
    pt = jnp.transpose(pred, (0, 2, 1))
    st = jnp.transpose(score, (0, 2, 1))
    l3 = label.reshape(B * nb, 1, bn)
    out = pl.pallas_call(
        _qfocal_body,
        grid=(B, nb),
        in_specs=[
            pl.BlockSpec((1, C, bn), lambda b, i: (b, 0, i)),
            pl.BlockSpec((1, 1, bn), lambda b, i, _nb=nb: (b * _nb + i, 0, 0)),
            pl.BlockSpec((1, C, bn), lambda b, i: (b, 0, i)),
        ],
        out_specs=pl.BlockSpec((1, C, bn), lambda b, i: (b, 0, i)),
        out_shape=jax.ShapeDtypeStruct((B, C, N), jnp.float32),
        compiler_params=pltpu.CompilerParams(
            dimension_semantics=("parallel", "parallel"),
        ),
    )(pt, l3, st)
    return jnp.transpose(out, (0, 2, 1))


# speedup vs baseline: 1.6317x; 1.6317x over previous
import jax
import jax.numpy as jnp
from jax.experimental import pallas as pl
from jax.experimental.pallas import tpu as pltpu

_GAMMA = 1.5


def _qfocal_body(pred_ref, label_ref, score_ref, out_ref):
    x = pred_ref[0]
    sc = score_ref[0]
    lab = label_ref[0]

    e = jnp.exp(-jnp.abs(x))
    recip = 1.0 / (1.0 + e)
    sig = jnp.where(x >= 0.0, recip, e * recip)
    bce0 = jnp.maximum(x, 0.0) + jnp.log1p(e)
    neg = bce0 * sig * jnp.sqrt(sig)

    s = jnp.max(sc, axis=0, keepdims=True)
    d = jnp.abs(s - sig)
    pos = (bce0 - x * s) * d * jnp.sqrt(d)

    cid = jax.lax.broadcasted_iota(jnp.int32, x.shape, 0)
    m = cid == lab
    out_ref[0] = jnp.where(m, pos, neg)


def kernel(pred, label, score):
    B, N, C = pred.shape
    bn = 4096
    nb = N // bn
    pt = jnp.transpose(pred, (0, 2, 1))
    st = jnp.transpose(score, (0, 2, 1))
    l3 = label.reshape(B * nb, 1, bn)
    out = pl.pallas_call(
        _qfocal_body,
        grid=(B, nb),
        in_specs=[
            pl.BlockSpec((1, C, bn), lambda b, i: (b, 0, i)),
            pl.BlockSpec((1, 1, bn), lambda b, i, _nb=nb: (b * _nb + i, 0, 0)),
            pl.BlockSpec((1, C, bn), lambda b, i: (b, 0, i)),
        ],
        out_specs=pl.BlockSpec((1, C, bn), lambda b, i: (b, 0, i)),
        out_shape=jax.ShapeDtypeStruct((B, C, N), jnp.float32),
        compiler_params=pltpu.CompilerParams(
            dimension_semantics=("parallel", "parallel"),
        ),
    )(pt, l3, st)
    return jnp.transpose(out, (0, 2, 1))


# speedup vs baseline: 1.7774x; 1.0893x over previous
import jax
import jax.numpy as jnp
from jax.experimental import pallas as pl
from jax.experimental.pallas import tpu as pltpu

_GAMMA = 1.5


def _qfocal_body(pred_ref, label_ref, score_ref, out_ref):
    x = pred_ref[0]
    sc = score_ref[0]
    lab = label_ref[0]

    e = jnp.exp(-jnp.abs(x))
    recip = 1.0 / (1.0 + e)
    sig = jnp.where(x >= 0.0, recip, e * recip)
    bce0 = jnp.maximum(x, 0.0) + jnp.log1p(e)
    neg = bce0 * sig * jnp.sqrt(sig)

    s = jnp.max(sc, axis=0, keepdims=True)
    d = jnp.abs(s - sig)
    pos = (bce0 - x * s) * d * jnp.sqrt(d)

    cid = jax.lax.broadcasted_iota(jnp.int32, x.shape, 0)
    m = cid == lab
    out_ref[0] = jnp.where(m, pos, neg)


def kernel(pred, label, score):
    B, N, C = pred.shape
    bn = 8192
    nb = N // bn
    pt = jnp.transpose(pred, (0, 2, 1))
    st = jnp.transpose(score, (0, 2, 1))
    l3 = label.reshape(B * nb, 1, bn)
    out = pl.pallas_call(
        _qfocal_body,
        grid=(B, nb),
        in_specs=[
            pl.BlockSpec((1, C, bn), lambda b, i: (b, 0, i)),
            pl.BlockSpec((1, 1, bn), lambda b, i, _nb=nb: (b * _nb + i, 0, 0)),
            pl.BlockSpec((1, C, bn), lambda b, i: (b, 0, i)),
        ],
        out_specs=pl.BlockSpec((1, C, bn), lambda b, i: (b, 0, i)),
        out_shape=jax.ShapeDtypeStruct((B, C, N), jnp.float32),
        compiler_params=pltpu.CompilerParams(
            dimension_semantics=("parallel", "parallel"),
        ),
    )(pt, l3, st)
    return jnp.transpose(out, (0, 2, 1))


# speedup vs baseline: 2.2584x; 1.2707x over previous
import jax
import jax.numpy as jnp
from jax.experimental import pallas as pl
from jax.experimental.pallas import tpu as pltpu

_GAMMA = 1.5


def _qfocal_body(pred_ref, label_ref, score_ref, out_ref):
    x = pred_ref[0]
    sc = score_ref[0]
    lab = label_ref[0]

    t = jnp.exp2(jnp.minimum(x * -1.4426950408889634, 126.0))
    d1 = 1.0 + t
    sig = 1.0 / d1
    bce0 = jnp.maximum(jnp.log(d1) + x, 0.0)

    s = jnp.max(sc, axis=0, keepdims=True)
    cid = jax.lax.broadcasted_iota(jnp.int32, x.shape, 0)
    m = cid == lab
    a = jnp.where(m, bce0 - x * s, bce0)
    b = jnp.where(m, jnp.abs(s - sig), sig)
    out_ref[0] = a * b * jnp.sqrt(b)


def kernel(pred, label, score):
    B, N, C = pred.shape
    bn = 8192
    nb = N // bn
    pt = jnp.transpose(pred, (0, 2, 1))
    st = jnp.transpose(score, (0, 2, 1))
    l3 = label.reshape(B * nb, 1, bn)
    out = pl.pallas_call(
        _qfocal_body,
        grid=(B, nb),
        in_specs=[
            pl.BlockSpec((1, C, bn), lambda b, i: (b, 0, i)),
            pl.BlockSpec((1, 1, bn), lambda b, i, _nb=nb: (b * _nb + i, 0, 0)),
            pl.BlockSpec((1, C, bn), lambda b, i: (b, 0, i)),
        ],
        out_specs=pl.BlockSpec((1, C, bn), lambda b, i: (b, 0, i)),
        out_shape=jax.ShapeDtypeStruct((B, C, N), jnp.float32),
        compiler_params=pltpu.CompilerParams(
            dimension_semantics=("parallel", "parallel"),
        ),
    )(pt, l3, st)
    return jnp.transpose(out, (0, 2, 1))


# speedup vs baseline: 2.3843x; 1.0558x over previous
import jax
import jax.numpy as jnp
from jax.experimental import pallas as pl
from jax.experimental.pallas import tpu as pltpu

_GAMMA = 1.5


def _qfocal_body(pred_ref, label_ref, score_ref, out_ref):
    x = pred_ref[...]
    sc = score_ref[...]
    lab = label_ref[...]

    t = jnp.exp2(jnp.minimum(x * -1.4426950408889634, 126.0))
    d1 = 1.0 + t
    sig = 1.0 / d1
    bce0 = jnp.maximum(jnp.log(d1) + x, 0.0)

    s = jnp.max(sc, axis=1, keepdims=True)
    cid = jax.lax.broadcasted_iota(jnp.int32, x.shape, 1)
    m = cid == lab
    a = jnp.where(m, bce0 - x * s, bce0)
    b = jnp.where(m, jnp.abs(s - sig), sig)
    out_ref[...] = a * b * jnp.sqrt(b)


def kernel(pred, label, score):
    B, N, C = pred.shape
    bb = 2
    pt = jnp.transpose(pred, (0, 2, 1))
    st = jnp.transpose(score, (0, 2, 1))
    l3 = label.reshape(B, 1, N)
    out = pl.pallas_call(
        _qfocal_body,
        grid=(B // bb,),
        in_specs=[
            pl.BlockSpec((bb, C, N), lambda i: (i, 0, 0)),
            pl.BlockSpec((bb, 1, N), lambda i: (i, 0, 0)),
            pl.BlockSpec((bb, C, N), lambda i: (i, 0, 0)),
        ],
        out_specs=pl.BlockSpec((bb, C, N), lambda i: (i, 0, 0)),
        out_shape=jax.ShapeDtypeStruct((B, C, N), jnp.float32),
        compiler_params=pltpu.CompilerParams(
            dimension_semantics=("parallel",),
        ),
    )(pt, l3, st)
    return jnp.transpose(out, (0, 2, 1))
